# baseline (speedup 1.0000x reference)
import functools

import jax
import jax.numpy as jnp
from jax import lax
from jax.experimental import pallas as pl
from jax.experimental.pallas import tpu as pltpu
from jax.experimental.pallas import tpu_sc as plsc

_B = 16384
_D = 128
_NC = 2
_NS = 16
_NW = _NC * _NS
_BPW = _B // _NW
_CH = 128
_NCHUNK = _BPW // _CH
_LANES = 16
_UNROLL = _D // _LANES
_ROWS_PER_IT = 2
_SCALE = 0.003 / (2.0 * _B * _D)

_mesh = plsc.VectorSubcoreMesh(core_axis_name="c", subcore_axis_name="s")


@functools.partial(
    pl.kernel,
    mesh=_mesh,
    out_type=jax.ShapeDtypeStruct((_NW, _LANES), jnp.float32),
    scratch_types=[
        pltpu.VMEM((_NCHUNK, _CH), jnp.int32),
        pltpu.VMEM((3, _CH, _D), jnp.float32),
        pltpu.VMEM((3, _CH, _D), jnp.float32),
        pltpu.VMEM((_LANES,), jnp.float32),
        pltpu.SemaphoreType.DMA,
        pltpu.SemaphoreType.DMA,
        pltpu.SemaphoreType.DMA,
        pltpu.SemaphoreType.DMA,
        pltpu.SemaphoreType.DMA,
        pltpu.SemaphoreType.DMA,
    ],
)
def _center_loss_sc(feat_hbm, lbl_hbm, cent_hbm, out_hbm,
                    idx_v, feat_v, rows_v, part_v,
                    sg0, sg1, sg2, sf0, sf1, sf2):
    c = lax.axis_index("c")
    s = lax.axis_index("s")
    w = s * _NC + c

    gsems = (sg0, sg1, sg2)
    fsems = (sf0, sf1, sf2)

    def start(j, slot):
        g = pltpu.async_copy(cent_hbm.at[idx_v.at[j]], rows_v.at[slot], gsems[slot])
        f = pltpu.async_copy(
            feat_hbm.at[pl.ds(w * _BPW + j * _CH, _CH)], feat_v.at[slot], fsems[slot]
        )
        return g, f

    f0 = pltpu.async_copy(feat_hbm.at[pl.ds(w * _BPW, _CH)], feat_v.at[0], fsems[0])
    f1 = pltpu.async_copy(feat_hbm.at[pl.ds(w * _BPW + _CH, _CH)], feat_v.at[1], fsems[1])
    pltpu.sync_copy(lbl_hbm.at[pl.ds(w * _NCHUNK, _NCHUNK)], idx_v)
    g0 = pltpu.async_copy(cent_hbm.at[idx_v.at[0]], rows_v.at[0], gsems[0])
    g1 = pltpu.async_copy(cent_hbm.at[idx_v.at[1]], rows_v.at[1], gsems[1])
    inflight = [(g0, f0), (g1, f1)]

    accs = tuple(jnp.zeros((_LANES,), jnp.float32) for _ in range(_UNROLL))

    for j in range(_NCHUNK):
        slot = j % 3
        pending = inflight.pop(0)
        pending[0].wait()
        pending[1].wait()
        if j + 2 < _NCHUNK:
            inflight.append(start(j + 2, (j + 2) % 3))
        fbuf = feat_v.at[slot]
        cbuf = rows_v.at[slot]

        @plsc.parallel_loop(0, _CH, step=1, unroll=4, carry=accs)
        def accs(row, acc, fbuf=fbuf, cbuf=cbuf):
            out = list(acc)
            for u in range(_UNROLL):
                fv = fbuf[row, pl.ds(u * _LANES, _LANES)]
                cv = cbuf[row, pl.ds(u * _LANES, _LANES)]
                d = fv - cv
                out[u] = out[u] + d * d
            return tuple(out)

    total = accs[0]
    for u in range(1, _UNROLL):
        total = total + accs[u]
    part_v[...] = total * _SCALE

    pltpu.sync_copy(part_v, out_hbm.at[w])


def kernel(features, labels, centers):
    lbl = labels.reshape(-1).astype(jnp.int32).reshape(_B // _CH, _CH)
    out = _center_loss_sc(features, lbl, centers)
    return jnp.sum(out)


# speedup vs baseline: 1.0086x; 1.0086x over previous
import functools

import jax
import jax.numpy as jnp
from jax import lax
from jax.experimental import pallas as pl
from jax.experimental.pallas import tpu as pltpu
from jax.experimental.pallas import tpu_sc as plsc

_B = 16384
_D = 128
_NC = 2
_NS = 16
_NW = _NC * _NS
_BPW = _B // _NW
_CH = 64
_NCHUNK = _BPW // _CH
_LANES = 16
_UNROLL = _D // _LANES
_ROWS_PER_IT = 2
_SCALE = 0.003 / (2.0 * _B * _D)

_mesh = plsc.VectorSubcoreMesh(core_axis_name="c", subcore_axis_name="s")


@functools.partial(
    pl.kernel,
    mesh=_mesh,
    out_type=jax.ShapeDtypeStruct((_NW, _LANES), jnp.float32),
    scratch_types=[
        pltpu.VMEM((_NCHUNK, _CH), jnp.int32),
        pltpu.VMEM((4, _CH, _D), jnp.float32),
        pltpu.VMEM((4, _CH, _D), jnp.float32),
        pltpu.VMEM((_LANES,), jnp.float32),
        pltpu.SemaphoreType.DMA,
        pltpu.SemaphoreType.DMA,
        pltpu.SemaphoreType.DMA,
        pltpu.SemaphoreType.DMA,
        pltpu.SemaphoreType.DMA,
        pltpu.SemaphoreType.DMA,
        pltpu.SemaphoreType.DMA,
        pltpu.SemaphoreType.DMA,
    ],
)
def _center_loss_sc(feat_hbm, lbl_hbm, cent_hbm, out_hbm,
                    idx_v, feat_v, rows_v, part_v,
                    sg0, sg1, sg2, sg3, sf0, sf1, sf2, sf3):
    c = lax.axis_index("c")
    s = lax.axis_index("s")
    w = s * _NC + c

    gsems = (sg0, sg1, sg2, sg3)
    fsems = (sf0, sf1, sf2, sf3)

    def start(j, slot):
        g = pltpu.async_copy(cent_hbm.at[idx_v.at[j]], rows_v.at[slot], gsems[slot])
        f = pltpu.async_copy(
            feat_hbm.at[pl.ds(w * _BPW + j * _CH, _CH)], feat_v.at[slot], fsems[slot]
        )
        return g, f

    fpre = [
        pltpu.async_copy(
            feat_hbm.at[pl.ds(w * _BPW + j * _CH, _CH)], feat_v.at[j], fsems[j]
        )
        for j in range(3)
    ]
    pltpu.sync_copy(lbl_hbm.at[pl.ds(w * _NCHUNK, _NCHUNK)], idx_v)
    inflight = [
        (pltpu.async_copy(cent_hbm.at[idx_v.at[j]], rows_v.at[j], gsems[j]), fpre[j])
        for j in range(3)
    ]

    accs = tuple(jnp.zeros((_LANES,), jnp.float32) for _ in range(_UNROLL))

    for j in range(_NCHUNK):
        slot = j % 4
        pending = inflight.pop(0)
        pending[0].wait()
        pending[1].wait()
        if j + 3 < _NCHUNK:
            inflight.append(start(j + 3, (j + 3) % 4))
        fbuf = feat_v.at[slot]
        cbuf = rows_v.at[slot]

        @plsc.parallel_loop(0, _CH, step=1, unroll=4, carry=accs)
        def accs(row, acc, fbuf=fbuf, cbuf=cbuf):
            out = list(acc)
            for u in range(_UNROLL):
                fv = fbuf[row, pl.ds(u * _LANES, _LANES)]
                cv = cbuf[row, pl.ds(u * _LANES, _LANES)]
                d = fv - cv
                out[u] = out[u] + d * d
            return tuple(out)

    total = accs[0]
    for u in range(1, _UNROLL):
        total = total + accs[u]
    part_v[...] = total * _SCALE

    pltpu.sync_copy(part_v, out_hbm.at[w])


def kernel(features, labels, centers):
    lbl = labels.reshape(-1).astype(jnp.int32).reshape(_B // _CH, _CH)
    out = _center_loss_sc(features, lbl, centers)
    return jnp.sum(out)
